# baseline (speedup 1.0000x reference)
import jax
import jax.numpy as jnp
from jax import lax
from jax.experimental import pallas as pl
from jax.experimental.pallas import tpu as pltpu
from jax.experimental.pallas import tpu_sc as plsc

_VOCAB = 1000000
_FEATURES = 32
_BATCH = 16384
_HIST = 50

_NC = 2
_NS = 16
_NW = _NC * _NS
_BPW = _BATCH // _NW
_CHUNK = 128
_NCHUNK = _BPW // _CHUNK
_LANES = 16
_STAGERS = 8
_COLCHUNK = _VOCAB // _STAGERS


def _sc_body_trivial(idx_hbm, col_hbm, out_hbm, col_s, idx_v, vals_v, out_v, sem):
  cid = lax.axis_index("c")
  sid = lax.axis_index("s")
  wid = sid * _NC + cid
  b0 = wid * _BPW
  @pl.loop(0, _BPW // _LANES)
  def _zero(g):
    out_v[pl.ds(g * _LANES, _LANES)] = jnp.zeros((_LANES,), jnp.float32)
  pltpu.sync_copy(out_v, out_hbm.at[pl.ds(b0, _BPW)])


def _sc_body(idx_hbm, col_hbm, out_hbm, col_s, idx_v, vals_v, out_v, sem):
  cid = lax.axis_index("c")
  sid = lax.axis_index("s")
  wid = sid * _NC + cid
  b0 = wid * _BPW

  @pl.when(sid < _STAGERS)
  def _stage_col():
    off = sid * _COLCHUNK
    pltpu.sync_copy(
        col_hbm.at[pl.ds(off, _COLCHUNK)], col_s.at[pl.ds(off, _COLCHUNK)]
    )

  pltpu.sync_copy(idx_hbm.at[:, pl.ds(b0, _BPW)], idx_v)
  plsc.subcore_barrier()

  @pl.loop(0, _HIST)
  def _gather(l):
    descs = []
    for c in range(_NCHUNK):
      sl = pl.ds(c * _CHUNK, _CHUNK)
      descs.append(
          pltpu.async_copy(col_s.at[idx_v.at[l, sl]], vals_v.at[l, sl], sem)
      )
    for d in descs:
      d.wait()

  @pl.loop(0, _BPW // _LANES)
  def _reduce(g):
    sl = pl.ds(g * _LANES, _LANES)
    acc = jnp.zeros((_LANES,), jnp.float32)
    for l in range(_HIST):
      acc = acc + vals_v[l, sl]
    out_v[sl] = acc * (1.0 / _HIST)

  pltpu.sync_copy(out_v, out_hbm.at[pl.ds(b0, _BPW)])


@jax.jit
def _sc_embedding_mean(idx_t, col0):
  mesh = plsc.VectorSubcoreMesh(core_axis_name="c", subcore_axis_name="s")
  return pl.kernel(
      _sc_body_trivial,
      out_type=jax.ShapeDtypeStruct((_BATCH,), jnp.float32),
      mesh=mesh,
      compiler_params=pltpu.CompilerParams(
          needs_layout_passes=False, use_tc_tiling_on_sc=False
      ),
      scratch_types=[
          pltpu.VMEM_SHARED((_VOCAB,), jnp.float32),
          pltpu.VMEM((_HIST, _BPW), jnp.int32),
          pltpu.VMEM((_HIST, _BPW), jnp.float32),
          pltpu.VMEM((_BPW,), jnp.float32),
          pltpu.SemaphoreType.DMA,
      ],
  )(idx_t, col0)


def kernel(index, table):
  idx_t = index.T.astype(jnp.int32)
  col0 = table[:, 0:1]
  out = _sc_embedding_mean(idx_t, col0)
  return out.reshape(_BATCH, 1)


# speedup vs baseline: 8.0948x; 8.0948x over previous
import jax
import jax.numpy as jnp
from jax import lax
from jax.experimental import pallas as pl
from jax.experimental.pallas import tpu as pltpu
from jax.experimental.pallas import tpu_sc as plsc

_VOCAB = 1000000
_FEATURES = 32
_BATCH = 16384
_HIST = 50

_NC = 2
_NS = 16
_NW = _NC * _NS
_BPW = _BATCH // _NW
_CHUNK = 128
_NCHUNK = _BPW // _CHUNK
_LANES = 16
_STAGERS = 8
_COLCHUNK = _VOCAB // _STAGERS

_XBLK = 8192


def _tc_extract_body(tab_ref, out_ref):
  out_ref[...] = tab_ref[0, :]


@jax.jit
def _tc_extract_col0(tab_t):
  grid = pl.cdiv(_VOCAB, _XBLK)
  return pl.pallas_call(
      _tc_extract_body,
      grid=(grid,),
      in_specs=[pl.BlockSpec((8, _XBLK), lambda j: (0, j))],
      out_specs=pl.BlockSpec((_XBLK,), lambda j: (j,)),
      out_shape=jax.ShapeDtypeStruct((_VOCAB,), jnp.float32),
  )(tab_t)


def _sc_body(idx_hbm, col_hbm, out_hbm, col_s, idx_v, vals_v, out_v, sem):
  cid = lax.axis_index("c")
  sid = lax.axis_index("s")
  wid = sid * _NC + cid
  b0 = wid * _BPW

  @pl.when(sid < _STAGERS)
  def _stage_col():
    off = sid * _COLCHUNK
    pltpu.sync_copy(
        col_hbm.at[pl.ds(off, _COLCHUNK)], col_s.at[pl.ds(off, _COLCHUNK)]
    )

  pltpu.sync_copy(idx_hbm.at[:, pl.ds(b0, _BPW)], idx_v)
  plsc.subcore_barrier()

  @pl.loop(0, _HIST)
  def _gather(l):
    descs = []
    for c in range(_NCHUNK):
      sl = pl.ds(c * _CHUNK, _CHUNK)
      descs.append(
          pltpu.async_copy(col_s.at[idx_v.at[l, sl]], vals_v.at[l, sl], sem)
      )
    for d in descs:
      d.wait()

  @pl.loop(0, _BPW // _LANES)
  def _reduce(g):
    sl = pl.ds(g * _LANES, _LANES)
    acc = jnp.zeros((_LANES,), jnp.float32)
    for l in range(_HIST):
      acc = acc + vals_v[l, sl]
    out_v[sl] = acc * (1.0 / _HIST)

  pltpu.sync_copy(out_v, out_hbm.at[pl.ds(b0, _BPW)])


@jax.jit
def _sc_embedding_mean(idx_t, col0):
  mesh = plsc.VectorSubcoreMesh(core_axis_name="c", subcore_axis_name="s")
  return pl.kernel(
      _sc_body,
      out_type=jax.ShapeDtypeStruct((_BATCH,), jnp.float32),
      mesh=mesh,
      compiler_params=pltpu.CompilerParams(
          needs_layout_passes=False, use_tc_tiling_on_sc=False
      ),
      scratch_types=[
          pltpu.VMEM_SHARED((_VOCAB,), jnp.float32),
          pltpu.VMEM((_HIST, _BPW), jnp.int32),
          pltpu.VMEM((_HIST, _BPW), jnp.float32),
          pltpu.VMEM((_BPW,), jnp.float32),
          pltpu.SemaphoreType.DMA,
      ],
  )(idx_t, col0)


def kernel(index, table):
  idx_t = index.T.astype(jnp.int32)
  col0 = _tc_extract_col0(table.T)
  out = _sc_embedding_mean(idx_t, col0)
  return out.reshape(_BATCH, 1)
